# Initial kernel scaffold; baseline (speedup 1.0000x reference)
import jax
import jax.numpy as jnp
from jax.experimental import pallas as pl


def kernel(drug_feature, drug_adj, ibatch, bn0_g, bn0_b, m1_w1, m1_b1, m1_w2, m1_b2, bn1_g, bn1_b, m2_w1, m2_b1, m2_w2, m2_b2, bn2_g, bn2_b, gcn1_w, gcn1_b, gcn2_w, gcn2_b):
    raise NotImplementedError("write your pallas kernel here")


# speedup vs baseline: 14.1969x; 14.1969x over previous
import functools

import jax
import jax.numpy as jnp
from jax import lax
from jax.experimental import pallas as pl
from jax.experimental.pallas import tpu as pltpu
from jax.experimental.pallas import tpu_sc as plsc

N = 10000
E = 320000
D = 128
H = 128
G = 256

NC = 2
NS = 16
NW = NC * NS
CH = 128
NCHUNK = 79
EPAD = NW * NCHUNK * CH
NPAD = 10240
RPT = NPAD // NS
RCH = RPT // CH
L = 16


@functools.cache
def _mesh():
    return plsc.VectorSubcoreMesh(core_axis_name="c", subcore_axis_name="s",
                                  num_cores=NC, num_subcores=NS)


def _zero_rows(ref, nrows, ncols):
    z = jnp.zeros((L,), jnp.float32)

    def row(i, _):
        def col(j, _):
            ref[i, pl.ds(j * L, L)] = z
            return 0
        return lax.fori_loop(0, ncols // L, col, 0)

    lax.fori_loop(0, nrows, row, 0)


def _zero_vec(ref, n):
    z = jnp.zeros((L,), jnp.float32)

    def it(i, _):
        ref[pl.ds(i * L, L)] = z
        return 0

    lax.fori_loop(0, n // L, it, 0)


def _scalar_edge_pass(idx_s, idx_d, a_vmem, acc, j):
    for cc in range(CH // L):
        isrc = idx_s[j, pl.ds(cc * L, L)]
        idst = idx_d[j, pl.ds(cc * L, L)]
        vals = plsc.load_gather(a_vmem, [isrc])
        plsc.addupdate_scatter(acc, [idst], vals)


def _sc_agg_body(x_hbm, src_hbm, dst_hbm, out128,
                 idx_s, idx_d, rows128, sem, spm128):
    c = lax.axis_index("c")
    s = lax.axis_index("s")
    wid = s * NC + c

    _zero_rows(rows128, CH, D)
    for k in range(RCH):
        r0 = s * RPT + k * CH
        pltpu.sync_copy(rows128, spm128.at[pl.ds(r0, CH)])
    plsc.subcore_barrier()

    pltpu.sync_copy(src_hbm.at[wid], idx_s)
    pltpu.sync_copy(dst_hbm.at[wid], idx_d)

    def chunk(j, _):
        pltpu.async_copy(x_hbm.at[idx_s.at[j]], rows128, sem).wait()
        pltpu.sync_copy(rows128, spm128.at[idx_d.at[j]], add=True)
        return 0

    lax.fori_loop(0, NCHUNK, chunk, 0)
    plsc.subcore_barrier()

    for k in range(RCH):
        r0 = s * RPT + k * CH
        pltpu.sync_copy(spm128.at[pl.ds(r0, CH)], rows128)
        pltpu.sync_copy(rows128, out128.at[c, pl.ds(r0, CH)])


@functools.cache
def _sc_agg_kernel():
  return pl.kernel(
    _sc_agg_body,
    out_type=jax.ShapeDtypeStruct((NC, NPAD, D), jnp.float32),
    mesh=_mesh(),
    compiler_params=pltpu.CompilerParams(needs_layout_passes=False),
    scratch_types=[
        pltpu.VMEM((NCHUNK, CH), jnp.int32),
        pltpu.VMEM((NCHUNK, CH), jnp.int32),
        pltpu.VMEM((CH, D), jnp.float32),
        pltpu.SemaphoreType.DMA,
        pltpu.VMEM_SHARED((NPAD, D), jnp.float32),
    ],
  )


def _sc_agg(x, src3, dst3):
    return _sc_agg_kernel()(x, src3, dst3)


def _sc_scal_body(a_hbm, src_hbm, dst_hbm, outs, idx_s, idx_d, a_vmem, acc):
    c = lax.axis_index("c")
    s = lax.axis_index("s")
    wid = s * NC + c

    _zero_vec(acc, NPAD)
    pltpu.sync_copy(src_hbm.at[wid], idx_s)
    pltpu.sync_copy(dst_hbm.at[wid], idx_d)
    pltpu.sync_copy(a_hbm, a_vmem)

    def chunk(j, _):
        _scalar_edge_pass(idx_s, idx_d, a_vmem, acc, j)
        return 0

    lax.fori_loop(0, NCHUNK, chunk, 0)
    pltpu.sync_copy(acc, outs.at[wid])


@functools.cache
def _sc_scal_kernel():
  return pl.kernel(
    _sc_scal_body,
    out_type=jax.ShapeDtypeStruct((NW, NPAD), jnp.float32),
    mesh=_mesh(),
    compiler_params=pltpu.CompilerParams(needs_layout_passes=False),
    scratch_types=[
        pltpu.VMEM((NCHUNK, CH), jnp.int32),
        pltpu.VMEM((NCHUNK, CH), jnp.int32),
        pltpu.VMEM((N,), jnp.float32),
        pltpu.VMEM((NPAD,), jnp.float32),
    ],
  )


def _sc_scal(a, src3, dst3):
    return _sc_scal_kernel()(a, src3, dst3)


def _bn(x, g, b):
    m = jnp.mean(x, axis=0, keepdims=True)
    v = jnp.mean((x - m) * (x - m), axis=0, keepdims=True)
    return (x - m) * lax.rsqrt(v + 1e-5) * g + b


def _tc_bn0_body(x_ref, g_ref, b_ref, o_ref):
    o_ref[...] = _bn(x_ref[...], g_ref[...], b_ref[...])


def _tc_bn0(x, g, b):
    return pl.pallas_call(
        _tc_bn0_body,
        out_shape=jax.ShapeDtypeStruct((N, D), jnp.float32),
    )(x, g.reshape(1, D), b.reshape(1, D))


def _gin_mlp(x, agg, w1, b1, w2, b2, bng, bnb):
    h = x + agg
    h = jnp.maximum(jnp.dot(h, w1, preferred_element_type=jnp.float32) + b1, 0.0)
    h = jnp.dot(h, w2, preferred_element_type=jnp.float32) + b2
    return _bn(jnp.maximum(h, 0.0), bng, bnb)


def _tc_gin1_body(x_ref, p_ref, pd_ref, w1_ref, b1_ref, w2_ref, b2_ref,
                  g_ref, bb_ref, gw_ref, x1_ref, a1_ref, dinv_ref):
    agg = p_ref[0, :N] + p_ref[1, :N]
    x1 = _gin_mlp(x_ref[...], agg, w1_ref[...], b1_ref[...], w2_ref[...],
                  b2_ref[...], g_ref[...], bb_ref[...])
    x1_ref[...] = x1
    deg = jnp.sum(pd_ref[:N], axis=1, keepdims=True) + 1.0
    dinv = lax.rsqrt(deg)
    dinv_ref[...] = dinv
    xw = jnp.dot(x1, gw_ref[...], preferred_element_type=jnp.float32)
    a1_ref[...] = xw * dinv


def _tc_gin1(x, p128, pdeg_t, w1, b1, w2, b2, bng, bnb, gw):
    return pl.pallas_call(
        _tc_gin1_body,
        out_shape=(
            jax.ShapeDtypeStruct((N, H), jnp.float32),
            jax.ShapeDtypeStruct((N, 1), jnp.float32),
            jax.ShapeDtypeStruct((N, 1), jnp.float32),
        ),
    )(x, p128, pdeg_t, w1, b1.reshape(1, 2 * H), w2, b2.reshape(1, H),
      bng.reshape(1, H), bnb.reshape(1, H), gw)


def _tc_gin2_body(x1_ref, p_ref, pg_ref, a1_ref, dinv_ref, w1_ref, b1_ref,
                  w2_ref, b2_ref, g_ref, bb_ref, gw_ref, gb_ref,
                  x2_ref, a2_ref, s1_ref):
    agg = p_ref[0, :N] + p_ref[1, :N]
    x2 = _gin_mlp(x1_ref[...], agg, w1_ref[...], b1_ref[...], w2_ref[...],
                  b2_ref[...], g_ref[...], bb_ref[...])
    x2_ref[...] = x2
    dinv = dinv_ref[...]
    xw = jnp.dot(x2, gw_ref[...], preferred_element_type=jnp.float32)
    a2_ref[...] = xw * dinv
    gsum = jnp.sum(pg_ref[:N], axis=1, keepdims=True)
    s1_ref[...] = jnp.tanh(dinv * (gsum + a1_ref[...]) + gb_ref[...])


def _tc_gin2(x1, p128, pg1_t, a1, dinv, w1, b1, w2, b2, bng, bnb, gw, gb):
    return pl.pallas_call(
        _tc_gin2_body,
        out_shape=(
            jax.ShapeDtypeStruct((N, H), jnp.float32),
            jax.ShapeDtypeStruct((N, 1), jnp.float32),
            jax.ShapeDtypeStruct((N, 1), jnp.float32),
        ),
    )(x1, p128, pg1_t, a1, dinv, w1, b1.reshape(1, 2 * H), w2,
      b2.reshape(1, H), bng.reshape(1, H), bnb.reshape(1, H), gw,
      gb.reshape(1, 1))


def _pool(m, s, x):
    smax = jnp.max(jnp.where(m > 0.0, s, -1e30), axis=0, keepdims=True)
    smax = jnp.where(smax > -1e29, smax, 0.0)
    sn = jnp.sum(m * smax, axis=1, keepdims=True)
    e = jnp.exp(s - sn)
    den = jnp.sum(m * e, axis=0, keepdims=True)
    denn = jnp.sum(m * den, axis=1, keepdims=True)
    ns = e / denn
    return lax.dot_general(m, x * ns, (((0,), (0,)), ((), ())),
                           preferred_element_type=jnp.float32)


def _tc_pool_body(x1_ref, x2_ref, s1_ref, pg_ref, a2_ref, dinv_ref,
                  gb_ref, ib_ref, g1_ref, g2_ref):
    dinv = dinv_ref[...]
    gsum = jnp.sum(pg_ref[:N], axis=1, keepdims=True)
    s2 = jnp.tanh(dinv * (gsum + a2_ref[...]) + gb_ref[...])
    gid = lax.broadcasted_iota(jnp.int32, (N, G), 1)
    m = (ib_ref[...] == gid).astype(jnp.float32)
    g1_ref[...] = _pool(m, s1_ref[...], x1_ref[...])
    g2_ref[...] = _pool(m, s2, x2_ref[...])


def _tc_pool(x1, x2, s1, pg2_t, a2, dinv, gb, ibatch):
    return pl.pallas_call(
        _tc_pool_body,
        compiler_params=pltpu.CompilerParams(
            vmem_limit_bytes=100 * 1024 * 1024),
        out_shape=(
            jax.ShapeDtypeStruct((G, H), jnp.float32),
            jax.ShapeDtypeStruct((G, H), jnp.float32),
        ),
    )(x1, x2, s1, pg2_t, a2, dinv, gb.reshape(1, 1),
      ibatch.reshape(N, 1))


def kernel(drug_feature, drug_adj, ibatch, bn0_g, bn0_b, m1_w1, m1_b1,
           m1_w2, m1_b2, bn1_g, bn1_b, m2_w1, m2_b1, m2_w2, m2_b2,
           bn2_g, bn2_b, gcn1_w, gcn1_b, gcn2_w, gcn2_b):
    pad = EPAD - E
    src3 = jnp.concatenate(
        [drug_adj[0], jnp.zeros((pad,), jnp.int32)]).reshape(NW, NCHUNK, CH)
    dst3 = jnp.concatenate(
        [drug_adj[1], jnp.full((pad,), NPAD - 1, jnp.int32)]
    ).reshape(NW, NCHUNK, CH)
    ones1d = jnp.ones((N,), jnp.float32)

    x = _tc_bn0(drug_feature, bn0_g, bn0_b)
    p128 = _sc_agg(x, src3, dst3)
    pdeg = _sc_scal(ones1d, src3, dst3)
    x1, a1, dinv = _tc_gin1(x, p128, pdeg.T, m1_w1, m1_b1, m1_w2, m1_b2,
                            bn1_g, bn1_b, gcn1_w)
    p128b = _sc_agg(x1, src3, dst3)
    pg1 = _sc_scal(a1[:, 0], src3, dst3)
    x2, a2, s1 = _tc_gin2(x1, p128b, pg1.T, a1, dinv, m2_w1, m2_b1,
                          m2_w2, m2_b2, bn2_g, bn2_b, gcn2_w, gcn1_b)
    pg2 = _sc_scal(a2[:, 0], src3, dst3)
    g1, g2 = _tc_pool(x1, x2, s1, pg2.T, a2, dinv, gcn2_b, ibatch)
    return jnp.stack((g1, g2), axis=1)
